# baseline (speedup 1.0000x reference)
import functools
import jax
import jax.numpy as jnp
import numpy as np
from jax import lax
from jax.experimental import pallas as pl
from jax.experimental.pallas import tpu as pltpu
from jax.experimental.pallas import tpu_sc as plsc

N = 100000
E = 6400000
NP = 102400
ROWS = NP // 128
NW = 32
EPW = E // NW
C = 2000
NCHUNK = EPW // C

_mesh = plsc.VectorSubcoreMesh(core_axis_name="c", subcore_axis_name="s")
_cp = pltpu.CompilerParams(needs_layout_passes=False)


@functools.partial(
    pl.kernel,
    out_type=jax.ShapeDtypeStruct((NW, NP), jnp.float32),
    mesh=_mesh,
    compiler_params=_cp,
    scratch_types=[
        pltpu.VMEM((NP,), jnp.float32),
        pltpu.VMEM((C,), jnp.int32),
    ],
)
def _sc_degree(dst_hbm, zeros_hbm, out_hbm, acc_v, idx_v):
    cid = lax.axis_index("c")
    sid = lax.axis_index("s")
    wid = cid * 16 + sid
    pltpu.sync_copy(zeros_hbm, acc_v)
    ones16 = jnp.full((16,), 1.0, jnp.float32)
    base = wid * EPW

    def body(j, carry):
        pltpu.sync_copy(dst_hbm.at[pl.ds(base + j * C, C)], idx_v)

        def inner(k, c2):
            iv = idx_v[pl.ds(k * 16, 16)]
            plsc.addupdate_scatter(acc_v, [iv], ones16)
            return c2

        lax.fori_loop(0, C // 16, inner, 0, unroll=8)
        return carry

    lax.fori_loop(0, NCHUNK, body, 0)
    pltpu.sync_copy(acc_v, out_hbm.at[wid])


@functools.partial(
    pl.kernel,
    out_type=jax.ShapeDtypeStruct((2, NP), jnp.float32),
    mesh=_mesh,
    compiler_params=_cp,
    scratch_types=[
        pltpu.VMEM((NP,), jnp.float32),
        pltpu.VMEM((C,), jnp.int32),
        pltpu.VMEM((C,), jnp.int32),
        pltpu.VMEM((C,), jnp.float32),
        pltpu.VMEM_SHARED((NP,), jnp.float32),
    ],
)
def _sc_scatter1(src_hbm, dst_hbm, tab_hbm, zeros_hbm, out_hbm,
                 tab_v, sidx_v, didx_v, val_v, acc_sh):
    cid = lax.axis_index("c")
    sid = lax.axis_index("s")
    wid = cid * 16 + sid
    pltpu.sync_copy(tab_hbm, tab_v)

    @pl.when(sid == 0)
    def _():
        pltpu.sync_copy(zeros_hbm, acc_sh)

    plsc.subcore_barrier()
    base = wid * EPW

    def body(j, carry):
        pltpu.sync_copy(src_hbm.at[pl.ds(base + j * C, C)], sidx_v)
        pltpu.sync_copy(dst_hbm.at[pl.ds(base + j * C, C)], didx_v)

        def inner(k, c2):
            iv = sidx_v[pl.ds(k * 16, 16)]
            val_v[pl.ds(k * 16, 16)] = plsc.load_gather(tab_v, [iv])
            return c2

        lax.fori_loop(0, C // 16, inner, 0, unroll=8)
        pltpu.sync_copy(val_v, acc_sh.at[didx_v], add=True)
        return carry

    lax.fori_loop(0, NCHUNK, body, 0)
    plsc.subcore_barrier()

    @pl.when(sid == 0)
    def _():
        pltpu.sync_copy(acc_sh, out_hbm.at[cid])


@functools.partial(
    pl.kernel,
    out_type=(jax.ShapeDtypeStruct((2, NP), jnp.float32),
              jax.ShapeDtypeStruct((2, NP), jnp.float32)),
    mesh=_mesh,
    compiler_params=_cp,
    scratch_types=[
        pltpu.VMEM((NP,), jnp.float32),
        pltpu.VMEM((C,), jnp.int32),
        pltpu.VMEM((C,), jnp.int32),
        pltpu.VMEM((C,), jnp.float32),
        pltpu.VMEM((C,), jnp.float32),
        pltpu.VMEM_SHARED((NP,), jnp.float32),
        pltpu.VMEM_SHARED((NP,), jnp.float32),
    ],
)
def _sc_scatter2(src_hbm, dst_hbm, tab_hbm, zeros_hbm, outp_hbm, outm_hbm,
                 tab_v, sidx_v, didx_v, valp_v, valm_v, accp_sh, accm_sh):
    cid = lax.axis_index("c")
    sid = lax.axis_index("s")
    wid = cid * 16 + sid
    pltpu.sync_copy(tab_hbm, tab_v)

    @pl.when(sid == 0)
    def _():
        pltpu.sync_copy(zeros_hbm, accp_sh)
        pltpu.sync_copy(zeros_hbm, accm_sh)

    plsc.subcore_barrier()
    base = wid * EPW

    def body(j, carry):
        pltpu.sync_copy(src_hbm.at[pl.ds(base + j * C, C)], sidx_v)
        pltpu.sync_copy(dst_hbm.at[pl.ds(base + j * C, C)], didx_v)

        def inner(k, c2):
            iv = sidx_v[pl.ds(k * 16, 16)]
            g = plsc.load_gather(tab_v, [iv])
            valp_v[pl.ds(k * 16, 16)] = jnp.maximum(g, 0.0)
            valm_v[pl.ds(k * 16, 16)] = jnp.maximum(-g, 0.0)
            return c2

        lax.fori_loop(0, C // 16, inner, 0, unroll=8)
        pltpu.sync_copy(valp_v, accp_sh.at[didx_v], add=True)
        pltpu.sync_copy(valm_v, accm_sh.at[didx_v], add=True)
        return carry

    lax.fori_loop(0, NCHUNK, body, 0)
    plsc.subcore_barrier()

    @pl.when(sid == 0)
    def _():
        pltpu.sync_copy(accp_sh, outp_hbm.at[cid])
        pltpu.sync_copy(accm_sh, outm_hbm.at[cid])


def _tc1_body(d, xr, dinv_ref, u_ref):
    deg = jnp.sum(d[...], axis=0) + 1.0
    dinv = lax.rsqrt(deg)
    dinv_ref[...] = dinv
    u_ref[...] = dinv * xr[...]


def _tc2_body(s0, s1, u, dinv, t_ref):
    t_ref[...] = dinv[...] * dinv[...] * (s0[...] + s1[...] + u[...])


def _tc3_body(p0, p1, m0, m1, t, dinv, w1, w2, b2, wfc, bfc, out_ref):
    tt = t[...]
    dv = dinv[...]
    P = dv * (p0[...] + p1[...] + jnp.maximum(tt, 0.0))
    M = dv * (m0[...] + m1[...] + jnp.maximum(-tt, 0.0))
    w1v = w1[...]
    a2 = jnp.dot(jnp.maximum(w1v, 0.0), w2[...],
                 preferred_element_type=jnp.float32)
    c2 = jnp.dot(jnp.maximum(-w1v, 0.0), w2[...],
                 preferred_element_type=jnp.float32)
    row = lax.broadcasted_iota(jnp.int32, (ROWS, 128), 0)
    col = lax.broadcasted_iota(jnp.int32, (ROWS, 128), 1)
    valid = (row * 128 + col) < N
    b2v = b2[...]
    sums = []
    for j in range(64):
        z = P * a2[0, j] + M * c2[0, j] + b2v[0, j]
        sums.append(jnp.sum(jnp.where(valid & (z > 0), z, 0.0)))
    pooled = jnp.stack(sums).reshape(1, 64) * (1.0 / N)
    out_ref[...] = jnp.dot(pooled, wfc[...],
                           preferred_element_type=jnp.float32) + bfc[...]


_tc1 = pl.pallas_call(
    _tc1_body,
    out_shape=(jax.ShapeDtypeStruct((ROWS, 128), jnp.float32),
               jax.ShapeDtypeStruct((ROWS, 128), jnp.float32)),
)

_tc2 = pl.pallas_call(
    _tc2_body,
    out_shape=jax.ShapeDtypeStruct((ROWS, 128), jnp.float32),
)

_tc3 = pl.pallas_call(
    _tc3_body,
    out_shape=jax.ShapeDtypeStruct((1, 2), jnp.float32),
)


def kernel(x, edge_index, W1, b1, W2, b2, Wfc, bfc):
    src = edge_index[0]
    dst = edge_index[1]
    zeros_np = jnp.zeros((NP,), jnp.float32)
    xp = jnp.pad(x[:, 0], (0, NP - N))

    deg_parts = _sc_degree(dst, zeros_np)
    dinv, u = _tc1(deg_parts.reshape(NW, ROWS, 128), xp.reshape(ROWS, 128))

    s1_parts = _sc_scatter1(src, dst, u.reshape(NP), zeros_np)
    t = _tc2(s1_parts[0].reshape(ROWS, 128), s1_parts[1].reshape(ROWS, 128),
             u, dinv)

    p_parts, m_parts = _sc_scatter2(src, dst, t.reshape(NP), zeros_np)
    out = _tc3(p_parts[0].reshape(ROWS, 128), p_parts[1].reshape(ROWS, 128),
               m_parts[0].reshape(ROWS, 128), m_parts[1].reshape(ROWS, 128),
               t, dinv, W1, W2, b2.reshape(1, 64), Wfc, bfc.reshape(1, 2))
    return out


# speedup vs baseline: 1.0837x; 1.0837x over previous
import functools
import jax
import jax.numpy as jnp
import numpy as np
from jax import lax
from jax.experimental import pallas as pl
from jax.experimental.pallas import tpu as pltpu
from jax.experimental.pallas import tpu_sc as plsc

N = 100000
E = 6400000
NP = 102400
ROWS = NP // 128
NW = 32
EPW = E // NW
C = 2000
NCHUNK = EPW // C
NPAIR = NCHUNK // 2
C2 = 800
NPAIR2 = EPW // C2 // 2

_mesh = plsc.VectorSubcoreMesh(core_axis_name="c", subcore_axis_name="s")
_cp = pltpu.CompilerParams(needs_layout_passes=False)


@functools.partial(
    pl.kernel,
    out_type=jax.ShapeDtypeStruct((2, NP), jnp.float32),
    mesh=_mesh,
    compiler_params=_cp,
    scratch_types=[
        pltpu.VMEM((C,), jnp.int32),
        pltpu.VMEM((C,), jnp.float32),
        pltpu.VMEM_SHARED((NP,), jnp.float32),
    ],
)
def _sc_degree(dst_hbm, zeros_hbm, out_hbm, idx_v, ones_v, acc_sh):
    cid = lax.axis_index("c")
    sid = lax.axis_index("s")
    wid = cid * 16 + sid

    def init_ones(i, carry):
        ones_v[pl.ds(i * 16, 16)] = jnp.full((16,), 1.0, jnp.float32)
        return carry

    lax.fori_loop(0, C // 16, init_ones, 0)

    @pl.when(sid == 0)
    def _():
        pltpu.sync_copy(zeros_hbm, acc_sh)

    plsc.subcore_barrier()
    base = wid * EPW

    def body(j, carry):
        pltpu.sync_copy(dst_hbm.at[pl.ds(base + j * C, C)], idx_v)
        pltpu.sync_copy(ones_v, acc_sh.at[idx_v], add=True)
        return carry

    lax.fori_loop(0, NCHUNK, body, 0)
    plsc.subcore_barrier()

    @pl.when(sid == 0)
    def _():
        pltpu.sync_copy(acc_sh, out_hbm.at[cid])


@functools.partial(
    pl.kernel,
    out_type=jax.ShapeDtypeStruct((2, NP), jnp.float32),
    mesh=_mesh,
    compiler_params=_cp,
    scratch_types=[
        pltpu.VMEM((NP,), jnp.float32),
        pltpu.VMEM((C,), jnp.int32),
        pltpu.VMEM((C,), jnp.int32),
        pltpu.VMEM((C,), jnp.int32),
        pltpu.VMEM((C,), jnp.int32),
        pltpu.VMEM((C,), jnp.float32),
        pltpu.VMEM((C,), jnp.float32),
        pltpu.VMEM_SHARED((NP,), jnp.float32),
        pltpu.SemaphoreType.DMA,
        pltpu.SemaphoreType.DMA,
    ],
)
def _sc_scatter1(src_hbm, dst_hbm, tab_hbm, zeros_hbm, out_hbm,
                 tab_v, sA, sB, dA, dB, vA, vB, acc_sh, semA, semB):
    cid = lax.axis_index("c")
    sid = lax.axis_index("s")
    wid = cid * 16 + sid
    pltpu.sync_copy(tab_hbm, tab_v)

    @pl.when(sid == 0)
    def _():
        pltpu.sync_copy(zeros_hbm, acc_sh)

    plsc.subcore_barrier()
    base = wid * EPW

    def gather(sidx, val):
        def inner(k, c2):
            iv = sidx[pl.ds(k * 16, 16)]
            val[pl.ds(k * 16, 16)] = plsc.load_gather(tab_v, [iv])
            return c2

        lax.fori_loop(0, C // 16, inner, 0, unroll=8)

    def half(jp, sidx, didx, val, sem, off):
        c0 = base + (2 * jp + off) * C

        @pl.when(jp > 0)
        def _():
            pltpu.make_async_copy(val, acc_sh.at[didx], sem).wait()

        pltpu.sync_copy(src_hbm.at[pl.ds(c0, C)], sidx)
        pltpu.sync_copy(dst_hbm.at[pl.ds(c0, C)], didx)
        gather(sidx, val)
        pltpu.async_copy(val, acc_sh.at[didx], sem, add=True)

    def body(jp, carry):
        half(jp, sA, dA, vA, semA, 0)
        half(jp, sB, dB, vB, semB, 1)
        return carry

    lax.fori_loop(0, NPAIR, body, 0)
    pltpu.make_async_copy(vA, acc_sh.at[dA], semA).wait()
    pltpu.make_async_copy(vB, acc_sh.at[dB], semB).wait()
    plsc.subcore_barrier()

    @pl.when(sid == 0)
    def _():
        pltpu.sync_copy(acc_sh, out_hbm.at[cid])


@functools.partial(
    pl.kernel,
    out_type=(jax.ShapeDtypeStruct((2, NP), jnp.float32),
              jax.ShapeDtypeStruct((2, NP), jnp.float32)),
    mesh=_mesh,
    compiler_params=_cp,
    scratch_types=[
        pltpu.VMEM((NP,), jnp.float32),
        pltpu.VMEM((C2,), jnp.int32),
        pltpu.VMEM((C2,), jnp.int32),
        pltpu.VMEM((C2,), jnp.int32),
        pltpu.VMEM((C2,), jnp.int32),
        pltpu.VMEM((C2,), jnp.float32),
        pltpu.VMEM((C2,), jnp.float32),
        pltpu.VMEM((C2,), jnp.float32),
        pltpu.VMEM((C2,), jnp.float32),
        pltpu.VMEM_SHARED((NP,), jnp.float32),
        pltpu.VMEM_SHARED((NP,), jnp.float32),
        pltpu.SemaphoreType.DMA,
        pltpu.SemaphoreType.DMA,
        pltpu.SemaphoreType.DMA,
        pltpu.SemaphoreType.DMA,
    ],
)
def _sc_scatter2(src_hbm, dst_hbm, tab_hbm, zeros_hbm, outp_hbm, outm_hbm,
                 tab_v, sA, sB, dA, dB, vpA, vpB, vmA, vmB,
                 accp_sh, accm_sh, semPA, semMA, semPB, semMB):
    cid = lax.axis_index("c")
    sid = lax.axis_index("s")
    wid = cid * 16 + sid
    pltpu.sync_copy(tab_hbm, tab_v)

    @pl.when(sid == 0)
    def _():
        pltpu.sync_copy(zeros_hbm, accp_sh)
        pltpu.sync_copy(zeros_hbm, accm_sh)

    plsc.subcore_barrier()
    base = wid * EPW

    def gather(sidx, vp, vm):
        def inner(k, c2):
            iv = sidx[pl.ds(k * 16, 16)]
            g = plsc.load_gather(tab_v, [iv])
            vp[pl.ds(k * 16, 16)] = jnp.maximum(g, 0.0)
            vm[pl.ds(k * 16, 16)] = jnp.maximum(-g, 0.0)
            return c2

        lax.fori_loop(0, C2 // 16, inner, 0, unroll=8)

    def half_a(jp):
        c0 = base + (2 * jp) * C2
        pltpu.sync_copy(src_hbm.at[pl.ds(c0, C2)], sA)
        pltpu.sync_copy(dst_hbm.at[pl.ds(c0, C2)], dA)
        gather(sA, vpA, vmA)

        @pl.when(jp > 0)
        def _():
            pltpu.make_async_copy(vpB, accp_sh.at[dB], semPB).wait()
            pltpu.make_async_copy(vmB, accm_sh.at[dB], semMB).wait()

        pltpu.async_copy(vpA, accp_sh.at[dA], semPA, add=True)
        pltpu.async_copy(vmA, accm_sh.at[dA], semMA, add=True)

    def half_b(jp):
        c1 = base + (2 * jp + 1) * C2
        pltpu.sync_copy(src_hbm.at[pl.ds(c1, C2)], sB)
        pltpu.sync_copy(dst_hbm.at[pl.ds(c1, C2)], dB)
        gather(sB, vpB, vmB)
        pltpu.make_async_copy(vpA, accp_sh.at[dA], semPA).wait()
        pltpu.make_async_copy(vmA, accm_sh.at[dA], semMA).wait()
        pltpu.async_copy(vpB, accp_sh.at[dB], semPB, add=True)
        pltpu.async_copy(vmB, accm_sh.at[dB], semMB, add=True)

    def body(jp, carry):
        half_a(jp)
        half_b(jp)
        return carry

    lax.fori_loop(0, NPAIR2, body, 0)
    pltpu.make_async_copy(vpB, accp_sh.at[dB], semPB).wait()
    pltpu.make_async_copy(vmB, accm_sh.at[dB], semMB).wait()
    plsc.subcore_barrier()

    @pl.when(sid == 0)
    def _():
        pltpu.sync_copy(accp_sh, outp_hbm.at[cid])
        pltpu.sync_copy(accm_sh, outm_hbm.at[cid])


def _tc1_body(d0, d1, xr, dinv_ref, u_ref):
    deg = d0[...] + d1[...] + 1.0
    dinv = lax.rsqrt(deg)
    dinv_ref[...] = dinv
    u_ref[...] = dinv * xr[...]


def _tc2_body(s0, s1, u, dinv, t_ref):
    t_ref[...] = dinv[...] * dinv[...] * (s0[...] + s1[...] + u[...])


def _tc3_body(p0, p1, m0, m1, t, dinv, w1, w2, b2, wfc, bfc, out_ref):
    tt = t[...]
    dv = dinv[...]
    P = dv * (p0[...] + p1[...] + jnp.maximum(tt, 0.0))
    M = dv * (m0[...] + m1[...] + jnp.maximum(-tt, 0.0))
    w1v = w1[...]
    a2 = jnp.dot(jnp.maximum(w1v, 0.0), w2[...],
                 preferred_element_type=jnp.float32)
    c2 = jnp.dot(jnp.maximum(-w1v, 0.0), w2[...],
                 preferred_element_type=jnp.float32)
    row = lax.broadcasted_iota(jnp.int32, (ROWS, 128), 0)
    col = lax.broadcasted_iota(jnp.int32, (ROWS, 128), 1)
    valid = (row * 128 + col) < N
    b2v = b2[...]
    sums = []
    for j in range(64):
        z = P * a2[0, j] + M * c2[0, j] + b2v[0, j]
        sums.append(jnp.sum(jnp.where(valid & (z > 0), z, 0.0)))
    pooled = jnp.stack(sums).reshape(1, 64) * (1.0 / N)
    out_ref[...] = jnp.dot(pooled, wfc[...],
                           preferred_element_type=jnp.float32) + bfc[...]


_tc1 = pl.pallas_call(
    _tc1_body,
    out_shape=(jax.ShapeDtypeStruct((ROWS, 128), jnp.float32),
               jax.ShapeDtypeStruct((ROWS, 128), jnp.float32)),
)

_tc2 = pl.pallas_call(
    _tc2_body,
    out_shape=jax.ShapeDtypeStruct((ROWS, 128), jnp.float32),
)

_tc3 = pl.pallas_call(
    _tc3_body,
    out_shape=jax.ShapeDtypeStruct((1, 2), jnp.float32),
)


def kernel(x, edge_index, W1, b1, W2, b2, Wfc, bfc):
    zeros_np = jnp.zeros((NP,), jnp.float32)
    xp = jnp.pad(x[:, 0], (0, NP - N))

    src = edge_index[0]
    dst = edge_index[1]
    deg_parts = _sc_degree(dst, zeros_np)
    d0 = deg_parts[0].reshape(ROWS, 128)
    d1 = deg_parts[1].reshape(ROWS, 128)
    dinv, u = _tc1(d0, d1, xp.reshape(ROWS, 128))

    s1_parts = _sc_scatter1(src, dst, u.reshape(NP), zeros_np)
    t = _tc2(s1_parts[0].reshape(ROWS, 128), s1_parts[1].reshape(ROWS, 128),
             u, dinv)

    p_parts, m_parts = _sc_scatter2(src, dst, t.reshape(NP), zeros_np)
    out = _tc3(p_parts[0].reshape(ROWS, 128), p_parts[1].reshape(ROWS, 128),
               m_parts[0].reshape(ROWS, 128), m_parts[1].reshape(ROWS, 128),
               t, dinv, W1, W2, b2.reshape(1, 64), Wfc, bfc.reshape(1, 2))
    return out


# speedup vs baseline: 1.0844x; 1.0006x over previous
import functools
import jax
import jax.numpy as jnp
import numpy as np
from jax import lax
from jax.experimental import pallas as pl
from jax.experimental.pallas import tpu as pltpu
from jax.experimental.pallas import tpu_sc as plsc

N = 100000
E = 6400000
NP = 102400
ROWS = NP // 128
NW = 32
EPW = E // NW
C = 2000
NCHUNK = EPW // C
NPAIR = NCHUNK // 2
C2 = 800
NPAIR2 = EPW // C2 // 2

_mesh = plsc.VectorSubcoreMesh(core_axis_name="c", subcore_axis_name="s")
_cp = pltpu.CompilerParams(needs_layout_passes=False)


@functools.partial(
    pl.kernel,
    out_type=jax.ShapeDtypeStruct((2, NP), jnp.float32),
    mesh=_mesh,
    compiler_params=_cp,
    scratch_types=[
        pltpu.VMEM((C,), jnp.int32),
        pltpu.VMEM((C,), jnp.float32),
        pltpu.VMEM_SHARED((NP,), jnp.float32),
    ],
)
def _sc_degree(dst_hbm, zeros_hbm, out_hbm, idx_v, ones_v, acc_sh):
    cid = lax.axis_index("c")
    sid = lax.axis_index("s")
    wid = cid * 16 + sid

    def init_ones(i, carry):
        ones_v[pl.ds(i * 16, 16)] = jnp.full((16,), 1.0, jnp.float32)
        return carry

    lax.fori_loop(0, C // 16, init_ones, 0)

    @pl.when(sid == 0)
    def _():
        pltpu.sync_copy(zeros_hbm, acc_sh)

    plsc.subcore_barrier()
    base = wid * EPW

    def body(j, carry):
        pltpu.sync_copy(dst_hbm.at[pl.ds(base + j * C, C)], idx_v)
        pltpu.sync_copy(ones_v, acc_sh.at[idx_v], add=True)
        return carry

    lax.fori_loop(0, NCHUNK, body, 0)
    plsc.subcore_barrier()

    @pl.when(sid == 0)
    def _():
        pltpu.sync_copy(acc_sh, out_hbm.at[cid])


@functools.partial(
    pl.kernel,
    out_type=jax.ShapeDtypeStruct((2, NP), jnp.float32),
    mesh=_mesh,
    compiler_params=_cp,
    scratch_types=[
        pltpu.VMEM((NP,), jnp.float32),
        pltpu.VMEM((C,), jnp.int32),
        pltpu.VMEM((C,), jnp.int32),
        pltpu.VMEM((C,), jnp.int32),
        pltpu.VMEM((C,), jnp.int32),
        pltpu.VMEM((C,), jnp.float32),
        pltpu.VMEM((C,), jnp.float32),
        pltpu.VMEM_SHARED((NP,), jnp.float32),
        pltpu.SemaphoreType.DMA,
        pltpu.SemaphoreType.DMA,
    ],
)
def _sc_scatter1(src_hbm, dst_hbm, tab_hbm, zeros_hbm, out_hbm,
                 tab_v, sA, sB, dA, dB, vA, vB, acc_sh, semA, semB):
    cid = lax.axis_index("c")
    sid = lax.axis_index("s")
    wid = cid * 16 + sid
    pltpu.sync_copy(tab_hbm, tab_v)

    @pl.when(sid == 0)
    def _():
        pltpu.sync_copy(zeros_hbm, acc_sh)

    plsc.subcore_barrier()
    base = wid * EPW

    def gather(sidx, val):
        def inner(k, c2):
            iv = sidx[pl.ds(k * 16, 16)]
            val[pl.ds(k * 16, 16)] = plsc.load_gather(tab_v, [iv])
            return c2

        lax.fori_loop(0, C // 16, inner, 0, unroll=8)

    def half(jp, sidx, didx, val, sem, off):
        c0 = base + (2 * jp + off) * C

        @pl.when(jp > 0)
        def _():
            pltpu.make_async_copy(val, acc_sh.at[didx], sem).wait()

        pltpu.sync_copy(src_hbm.at[pl.ds(c0, C)], sidx)
        pltpu.sync_copy(dst_hbm.at[pl.ds(c0, C)], didx)
        gather(sidx, val)
        pltpu.async_copy(val, acc_sh.at[didx], sem, add=True)

    def body(jp, carry):
        half(jp, sA, dA, vA, semA, 0)
        half(jp, sB, dB, vB, semB, 1)
        return carry

    lax.fori_loop(0, NPAIR, body, 0)
    pltpu.make_async_copy(vA, acc_sh.at[dA], semA).wait()
    pltpu.make_async_copy(vB, acc_sh.at[dB], semB).wait()
    plsc.subcore_barrier()

    @pl.when(sid == 0)
    def _():
        pltpu.sync_copy(acc_sh, out_hbm.at[cid])


@functools.partial(
    pl.kernel,
    out_type=(jax.ShapeDtypeStruct((2, NP), jnp.float32),
              jax.ShapeDtypeStruct((2, NP), jnp.float32)),
    mesh=_mesh,
    compiler_params=_cp,
    scratch_types=[
        pltpu.VMEM((NP,), jnp.float32),
        pltpu.VMEM((C2,), jnp.int32),
        pltpu.VMEM((C2,), jnp.int32),
        pltpu.VMEM((C2,), jnp.int32),
        pltpu.VMEM((C2,), jnp.int32),
        pltpu.VMEM((C2,), jnp.float32),
        pltpu.VMEM((C2,), jnp.float32),
        pltpu.VMEM((C2,), jnp.float32),
        pltpu.VMEM((C2,), jnp.float32),
        pltpu.VMEM_SHARED((NP,), jnp.float32),
        pltpu.VMEM_SHARED((NP,), jnp.float32),
        pltpu.SemaphoreType.DMA,
        pltpu.SemaphoreType.DMA,
        pltpu.SemaphoreType.DMA,
        pltpu.SemaphoreType.DMA,
    ],
)
def _sc_scatter2(src_hbm, dst_hbm, tab_hbm, zeros_hbm, outp_hbm, outm_hbm,
                 tab_v, sA, sB, dA, dB, vpA, vpB, vmA, vmB,
                 accp_sh, accm_sh, semPA, semMA, semPB, semMB):
    cid = lax.axis_index("c")
    sid = lax.axis_index("s")
    wid = cid * 16 + sid
    pltpu.sync_copy(tab_hbm, tab_v)

    @pl.when(sid == 0)
    def _():
        pltpu.sync_copy(zeros_hbm, accp_sh)
        pltpu.sync_copy(zeros_hbm, accm_sh)

    plsc.subcore_barrier()
    base = wid * EPW

    def gather(sidx, vp, vm):
        def inner(k, c2):
            iv = sidx[pl.ds(k * 16, 16)]
            g = plsc.load_gather(tab_v, [iv])
            vp[pl.ds(k * 16, 16)] = jnp.maximum(g, 0.0)
            vm[pl.ds(k * 16, 16)] = jnp.maximum(-g, 0.0)
            return c2

        lax.fori_loop(0, C2 // 16, inner, 0, unroll=8)

    def half_a(jp):
        c0 = base + (2 * jp) * C2
        pltpu.sync_copy(src_hbm.at[pl.ds(c0, C2)], sA)
        pltpu.sync_copy(dst_hbm.at[pl.ds(c0, C2)], dA)
        gather(sA, vpA, vmA)

        @pl.when(jp > 0)
        def _():
            pltpu.make_async_copy(vpB, accp_sh.at[dB], semPB).wait()
            pltpu.make_async_copy(vmB, accm_sh.at[dB], semMB).wait()

        pltpu.async_copy(vpA, accp_sh.at[dA], semPA, add=True)
        pltpu.async_copy(vmA, accm_sh.at[dA], semMA, add=True)

    def half_b(jp):
        c1 = base + (2 * jp + 1) * C2
        pltpu.sync_copy(src_hbm.at[pl.ds(c1, C2)], sB)
        pltpu.sync_copy(dst_hbm.at[pl.ds(c1, C2)], dB)
        gather(sB, vpB, vmB)
        pltpu.make_async_copy(vpA, accp_sh.at[dA], semPA).wait()
        pltpu.make_async_copy(vmA, accm_sh.at[dA], semMA).wait()
        pltpu.async_copy(vpB, accp_sh.at[dB], semPB, add=True)
        pltpu.async_copy(vmB, accm_sh.at[dB], semMB, add=True)

    def body(jp, carry):
        half_a(jp)
        half_b(jp)
        return carry

    lax.fori_loop(0, NPAIR2, body, 0)
    pltpu.make_async_copy(vpB, accp_sh.at[dB], semPB).wait()
    pltpu.make_async_copy(vmB, accm_sh.at[dB], semMB).wait()
    plsc.subcore_barrier()

    @pl.when(sid == 0)
    def _():
        pltpu.sync_copy(accp_sh, outp_hbm.at[cid])
        pltpu.sync_copy(accm_sh, outm_hbm.at[cid])


def _tc1_body(d0, d1, xr, dinv_ref, u_ref):
    deg = d0[...] + d1[...] + 1.0
    dinv = lax.rsqrt(deg)
    dinv_ref[...] = dinv
    u_ref[...] = dinv * xr[...]


def _tc2_body(s0, s1, u, dinv, t_ref):
    t_ref[...] = dinv[...] * dinv[...] * (s0[...] + s1[...] + u[...])


def _tc3_body(p0, p1, m0, m1, t, dinv, w1, w2, b2, wfc, bfc, out_ref):
    tt = t[...]
    dv = dinv[...]
    P = dv * (p0[...] + p1[...] + jnp.maximum(tt, 0.0))
    M = dv * (m0[...] + m1[...] + jnp.maximum(-tt, 0.0))
    w1v = w1[...]
    a2 = jnp.dot(jnp.maximum(w1v, 0.0), w2[...],
                 preferred_element_type=jnp.float32)
    c2 = jnp.dot(jnp.maximum(-w1v, 0.0), w2[...],
                 preferred_element_type=jnp.float32)
    row = lax.broadcasted_iota(jnp.int32, (ROWS, 128), 0)
    col = lax.broadcasted_iota(jnp.int32, (ROWS, 128), 1)
    valid = (row * 128 + col) < N
    b2v = b2[...]
    sums = []
    for j in range(64):
        z = P * a2[0, j] + M * c2[0, j] + b2v[0, j]
        sums.append(jnp.sum(jnp.where(valid & (z > 0), z, 0.0)))
    pooled = jnp.stack(sums).reshape(1, 64) * (1.0 / N)
    out_ref[...] = jnp.dot(pooled, wfc[...],
                           preferred_element_type=jnp.float32) + bfc[...]


_tc1 = pl.pallas_call(
    _tc1_body,
    out_shape=(jax.ShapeDtypeStruct((ROWS, 128), jnp.float32),
               jax.ShapeDtypeStruct((ROWS, 128), jnp.float32)),
)

_tc2 = pl.pallas_call(
    _tc2_body,
    out_shape=jax.ShapeDtypeStruct((ROWS, 128), jnp.float32),
)

_tc3 = pl.pallas_call(
    _tc3_body,
    out_shape=jax.ShapeDtypeStruct((1, 2), jnp.float32),
)


def kernel(x, edge_index, W1, b1, W2, b2, Wfc, bfc):
    src = edge_index[0]
    dst = edge_index[1]
    zeros_np = jnp.zeros((NP,), jnp.float32)
    xp = jnp.pad(x[:, 0], (0, NP - N))

    deg_parts = _sc_degree(dst, zeros_np)
    d0 = deg_parts[0].reshape(ROWS, 128)
    d1 = deg_parts[1].reshape(ROWS, 128)
    dinv, u = _tc1(d0, d1, xp.reshape(ROWS, 128))

    s1_parts = _sc_scatter1(src, dst, u.reshape(NP), zeros_np)
    t = _tc2(s1_parts[0].reshape(ROWS, 128), s1_parts[1].reshape(ROWS, 128),
             u, dinv)

    p_parts, m_parts = _sc_scatter2(src, dst, t.reshape(NP), zeros_np)
    out = _tc3(p_parts[0].reshape(ROWS, 128), p_parts[1].reshape(ROWS, 128),
               m_parts[0].reshape(ROWS, 128), m_parts[1].reshape(ROWS, 128),
               t, dinv, W1, W2, b2.reshape(1, 64), Wfc, bfc.reshape(1, 2))
    return out


# speedup vs baseline: 1.1025x; 1.0167x over previous
import functools
import jax
import jax.numpy as jnp
from jax import lax
from jax.experimental import pallas as pl
from jax.experimental.pallas import tpu as pltpu
from jax.experimental.pallas import tpu_sc as plsc

N = 100000
E = 6400000
NP = 102400
ROWS = NP // 128
NW = 32
EPW = E // NW
C = 2000
NCHUNK = EPW // C
NPAIR = NCHUNK // 2
C2 = 800
NPAIR2 = EPW // C2 // 2

_mesh = plsc.VectorSubcoreMesh(core_axis_name="c", subcore_axis_name="s")
_cp = pltpu.CompilerParams(needs_layout_passes=False)


@functools.partial(
    pl.kernel,
    out_type=jax.ShapeDtypeStruct((2, NP), jnp.float32),
    mesh=_mesh,
    compiler_params=_cp,
    scratch_types=[
        pltpu.VMEM((C,), jnp.int32),
        pltpu.VMEM((C,), jnp.float32),
        pltpu.VMEM_SHARED((NP,), jnp.float32),
    ],
)
def _sc_degree(edges_hbm, zeros_hbm, out_hbm, idx_v, ones_v, acc_sh):
    cid = lax.axis_index("c")
    sid = lax.axis_index("s")
    wid = cid * 16 + sid

    def init_ones(i, carry):
        ones_v[pl.ds(i * 16, 16)] = jnp.full((16,), 1.0, jnp.float32)
        return carry

    lax.fori_loop(0, C // 16, init_ones, 0)

    @pl.when(sid == 0)
    def _():
        pltpu.sync_copy(zeros_hbm, acc_sh)

    plsc.subcore_barrier()
    base = wid * EPW

    def body(j, carry):
        pltpu.sync_copy(edges_hbm.at[pl.ds(E + base + j * C, C)], idx_v)
        pltpu.sync_copy(ones_v, acc_sh.at[idx_v], add=True)
        return carry

    lax.fori_loop(0, NCHUNK, body, 0)
    plsc.subcore_barrier()

    @pl.when(sid == 0)
    def _():
        pltpu.sync_copy(acc_sh, out_hbm.at[cid])


@functools.partial(
    pl.kernel,
    out_type=jax.ShapeDtypeStruct((2, NP), jnp.float32),
    mesh=_mesh,
    compiler_params=_cp,
    scratch_types=[
        pltpu.VMEM((NP,), jnp.float32),
        pltpu.VMEM((C,), jnp.int32),
        pltpu.VMEM((C,), jnp.int32),
        pltpu.VMEM((C,), jnp.int32),
        pltpu.VMEM((C,), jnp.int32),
        pltpu.VMEM((C,), jnp.float32),
        pltpu.VMEM((C,), jnp.float32),
        pltpu.VMEM_SHARED((NP,), jnp.float32),
        pltpu.SemaphoreType.DMA,
        pltpu.SemaphoreType.DMA,
    ],
)
def _sc_scatter1(edges_hbm, tab_hbm, zeros_hbm, out_hbm,
                 tab_v, sA, sB, dA, dB, vA, vB, acc_sh, semA, semB):
    cid = lax.axis_index("c")
    sid = lax.axis_index("s")
    wid = cid * 16 + sid
    pltpu.sync_copy(tab_hbm, tab_v)

    @pl.when(sid == 0)
    def _():
        pltpu.sync_copy(zeros_hbm, acc_sh)

    plsc.subcore_barrier()
    base = wid * EPW

    def gather(sidx, val):
        def inner(k, c2):
            iv = sidx[pl.ds(k * 16, 16)]
            val[pl.ds(k * 16, 16)] = plsc.load_gather(tab_v, [iv])
            return c2

        lax.fori_loop(0, C // 16, inner, 0, unroll=8)

    def half(jp, sidx, didx, val, sem, off):
        c0 = base + (2 * jp + off) * C

        @pl.when(jp > 0)
        def _():
            pltpu.make_async_copy(val, acc_sh.at[didx], sem).wait()

        pltpu.sync_copy(edges_hbm.at[pl.ds(c0, C)], sidx)
        pltpu.sync_copy(edges_hbm.at[pl.ds(E + c0, C)], didx)
        gather(sidx, val)
        pltpu.async_copy(val, acc_sh.at[didx], sem, add=True)

    def body(jp, carry):
        half(jp, sA, dA, vA, semA, 0)
        half(jp, sB, dB, vB, semB, 1)
        return carry

    lax.fori_loop(0, NPAIR, body, 0)
    pltpu.make_async_copy(vA, acc_sh.at[dA], semA).wait()
    pltpu.make_async_copy(vB, acc_sh.at[dB], semB).wait()
    plsc.subcore_barrier()

    @pl.when(sid == 0)
    def _():
        pltpu.sync_copy(acc_sh, out_hbm.at[cid])


@functools.partial(
    pl.kernel,
    out_type=(jax.ShapeDtypeStruct((2, NP), jnp.float32),
              jax.ShapeDtypeStruct((2, NP), jnp.float32)),
    mesh=_mesh,
    compiler_params=_cp,
    scratch_types=[
        pltpu.VMEM((NP,), jnp.float32),
        pltpu.VMEM((C2,), jnp.int32),
        pltpu.VMEM((C2,), jnp.int32),
        pltpu.VMEM((C2,), jnp.int32),
        pltpu.VMEM((C2,), jnp.int32),
        pltpu.VMEM((C2,), jnp.float32),
        pltpu.VMEM((C2,), jnp.float32),
        pltpu.VMEM((C2,), jnp.float32),
        pltpu.VMEM((C2,), jnp.float32),
        pltpu.VMEM_SHARED((NP,), jnp.float32),
        pltpu.VMEM_SHARED((NP,), jnp.float32),
        pltpu.SemaphoreType.DMA,
        pltpu.SemaphoreType.DMA,
        pltpu.SemaphoreType.DMA,
        pltpu.SemaphoreType.DMA,
    ],
)
def _sc_scatter2(edges_hbm, tab_hbm, zeros_hbm, outp_hbm, outm_hbm,
                 tab_v, sA, sB, dA, dB, vpA, vpB, vmA, vmB,
                 accp_sh, accm_sh, semPA, semMA, semPB, semMB):
    cid = lax.axis_index("c")
    sid = lax.axis_index("s")
    wid = cid * 16 + sid
    pltpu.sync_copy(tab_hbm, tab_v)

    @pl.when(sid == 0)
    def _():
        pltpu.sync_copy(zeros_hbm, accp_sh)
        pltpu.sync_copy(zeros_hbm, accm_sh)

    plsc.subcore_barrier()
    base = wid * EPW

    def gather(sidx, vp, vm):
        def inner(k, c2):
            iv = sidx[pl.ds(k * 16, 16)]
            g = plsc.load_gather(tab_v, [iv])
            vp[pl.ds(k * 16, 16)] = jnp.maximum(g, 0.0)
            vm[pl.ds(k * 16, 16)] = jnp.maximum(-g, 0.0)
            return c2

        lax.fori_loop(0, C2 // 16, inner, 0, unroll=8)

    def half_a(jp):
        c0 = base + (2 * jp) * C2
        pltpu.sync_copy(edges_hbm.at[pl.ds(c0, C2)], sA)
        pltpu.sync_copy(edges_hbm.at[pl.ds(E + c0, C2)], dA)
        gather(sA, vpA, vmA)

        @pl.when(jp > 0)
        def _():
            pltpu.make_async_copy(vpB, accp_sh.at[dB], semPB).wait()
            pltpu.make_async_copy(vmB, accm_sh.at[dB], semMB).wait()

        pltpu.async_copy(vpA, accp_sh.at[dA], semPA, add=True)
        pltpu.async_copy(vmA, accm_sh.at[dA], semMA, add=True)

    def half_b(jp):
        c1 = base + (2 * jp + 1) * C2
        pltpu.sync_copy(edges_hbm.at[pl.ds(c1, C2)], sB)
        pltpu.sync_copy(edges_hbm.at[pl.ds(E + c1, C2)], dB)
        gather(sB, vpB, vmB)
        pltpu.make_async_copy(vpA, accp_sh.at[dA], semPA).wait()
        pltpu.make_async_copy(vmA, accm_sh.at[dA], semMA).wait()
        pltpu.async_copy(vpB, accp_sh.at[dB], semPB, add=True)
        pltpu.async_copy(vmB, accm_sh.at[dB], semMB, add=True)

    def body(jp, carry):
        half_a(jp)
        half_b(jp)
        return carry

    lax.fori_loop(0, NPAIR2, body, 0)
    pltpu.make_async_copy(vpB, accp_sh.at[dB], semPB).wait()
    pltpu.make_async_copy(vmB, accm_sh.at[dB], semMB).wait()
    plsc.subcore_barrier()

    @pl.when(sid == 0)
    def _():
        pltpu.sync_copy(accp_sh, outp_hbm.at[cid])
        pltpu.sync_copy(accm_sh, outm_hbm.at[cid])


def _tc1_body(d0, d1, xr, dinv_ref, u_ref):
    deg = d0[...] + d1[...] + 1.0
    dinv = lax.rsqrt(deg)
    dinv_ref[...] = dinv
    u_ref[...] = dinv * xr[...]


def _tc2_body(s0, s1, u, dinv, t_ref):
    t_ref[...] = dinv[...] * dinv[...] * (s0[...] + s1[...] + u[...])


def _tc3_body(p0, p1, m0, m1, t, dinv, w1, w2, b2, wfc, bfc, out_ref):
    tt = t[...]
    dv = dinv[...]
    P = dv * (p0[...] + p1[...] + jnp.maximum(tt, 0.0))
    M = dv * (m0[...] + m1[...] + jnp.maximum(-tt, 0.0))
    w1v = w1[...]
    a2 = jnp.dot(jnp.maximum(w1v, 0.0), w2[...],
                 preferred_element_type=jnp.float32)
    c2 = jnp.dot(jnp.maximum(-w1v, 0.0), w2[...],
                 preferred_element_type=jnp.float32)
    row = lax.broadcasted_iota(jnp.int32, (ROWS, 128), 0)
    col = lax.broadcasted_iota(jnp.int32, (ROWS, 128), 1)
    valid = (row * 128 + col) < N
    b2v = b2[...]
    sums = []
    for j in range(64):
        z = P * a2[0, j] + M * c2[0, j] + b2v[0, j]
        sums.append(jnp.sum(jnp.where(valid & (z > 0), z, 0.0)))
    pooled = jnp.stack(sums).reshape(1, 64) * (1.0 / N)
    out_ref[...] = jnp.dot(pooled, wfc[...],
                           preferred_element_type=jnp.float32) + bfc[...]


_tc1 = pl.pallas_call(
    _tc1_body,
    out_shape=(jax.ShapeDtypeStruct((ROWS, 128), jnp.float32),
               jax.ShapeDtypeStruct((ROWS, 128), jnp.float32)),
)

_tc2 = pl.pallas_call(
    _tc2_body,
    out_shape=jax.ShapeDtypeStruct((ROWS, 128), jnp.float32),
)

_tc3 = pl.pallas_call(
    _tc3_body,
    out_shape=jax.ShapeDtypeStruct((1, 2), jnp.float32),
)


def kernel(x, edge_index, W1, b1, W2, b2, Wfc, bfc):
    edges = edge_index.reshape(2 * E)
    zeros_np = jnp.zeros((NP,), jnp.float32)
    xp = jnp.pad(x[:, 0], (0, NP - N))

    deg_parts = _sc_degree(edges, zeros_np)
    d0 = deg_parts[0].reshape(ROWS, 128)
    d1 = deg_parts[1].reshape(ROWS, 128)
    dinv, u = _tc1(d0, d1, xp.reshape(ROWS, 128))

    s1_parts = _sc_scatter1(edges, u.reshape(NP), zeros_np)
    t = _tc2(s1_parts[0].reshape(ROWS, 128), s1_parts[1].reshape(ROWS, 128),
             u, dinv)

    p_parts, m_parts = _sc_scatter2(edges, t.reshape(NP), zeros_np)
    out = _tc3(p_parts[0].reshape(ROWS, 128), p_parts[1].reshape(ROWS, 128),
               m_parts[0].reshape(ROWS, 128), m_parts[1].reshape(ROWS, 128),
               t, dinv, W1, W2, b2.reshape(1, 64), Wfc, bfc.reshape(1, 2))
    return out


# speedup vs baseline: 1.1033x; 1.0007x over previous
import functools
import jax
import jax.numpy as jnp
from jax import lax
from jax.experimental import pallas as pl
from jax.experimental.pallas import tpu as pltpu
from jax.experimental.pallas import tpu_sc as plsc

N = 100000
E = 6400000
NP = 102400
ROWS = NP // 128
NW = 32
EPW = E // NW
C = 2000
NCHUNK = EPW // C
NPAIR = NCHUNK // 2
C2 = 800
NPAIR2 = EPW // C2 // 2

_mesh = plsc.VectorSubcoreMesh(core_axis_name="c", subcore_axis_name="s")
_cp = pltpu.CompilerParams(needs_layout_passes=False)


@functools.partial(
    pl.kernel,
    out_type=jax.ShapeDtypeStruct((2, NP), jnp.float32),
    mesh=_mesh,
    compiler_params=_cp,
    scratch_types=[
        pltpu.VMEM((C,), jnp.int32),
        pltpu.VMEM((C,), jnp.float32),
        pltpu.VMEM_SHARED((NP,), jnp.float32),
    ],
)
def _sc_degree(edges_hbm, zeros_hbm, out_hbm, idx_v, ones_v, acc_sh):
    cid = lax.axis_index("c")
    sid = lax.axis_index("s")
    wid = cid * 16 + sid

    def init_ones(i, carry):
        ones_v[pl.ds(i * 16, 16)] = jnp.full((16,), 1.0, jnp.float32)
        return carry

    lax.fori_loop(0, C // 16, init_ones, 0)

    @pl.when(sid == 0)
    def _():
        pltpu.sync_copy(zeros_hbm, acc_sh)

    plsc.subcore_barrier()
    base = wid * EPW

    def body(j, carry):
        pltpu.sync_copy(edges_hbm.at[pl.ds(E + base + j * C, C)], idx_v)
        pltpu.sync_copy(ones_v, acc_sh.at[idx_v], add=True)
        return carry

    lax.fori_loop(0, NCHUNK, body, 0)
    plsc.subcore_barrier()

    @pl.when(sid == 0)
    def _():
        pltpu.sync_copy(acc_sh, out_hbm.at[cid])


@functools.partial(
    pl.kernel,
    out_type=jax.ShapeDtypeStruct((2, NP), jnp.float32),
    mesh=_mesh,
    compiler_params=_cp,
    scratch_types=[
        pltpu.VMEM((NP,), jnp.float32),
        pltpu.VMEM((C,), jnp.int32),
        pltpu.VMEM((C,), jnp.int32),
        pltpu.VMEM((C,), jnp.int32),
        pltpu.VMEM((C,), jnp.int32),
        pltpu.VMEM((C,), jnp.float32),
        pltpu.VMEM((C,), jnp.float32),
        pltpu.VMEM_SHARED((NP,), jnp.float32),
        pltpu.SemaphoreType.DMA,
        pltpu.SemaphoreType.DMA,
    ],
)
def _sc_scatter1(edges_hbm, tab_hbm, zeros_hbm, out_hbm,
                 tab_v, sA, sB, dA, dB, vA, vB, acc_sh, semA, semB):
    cid = lax.axis_index("c")
    sid = lax.axis_index("s")
    wid = cid * 16 + sid
    pltpu.sync_copy(tab_hbm, tab_v)

    @pl.when(sid == 0)
    def _():
        pltpu.sync_copy(zeros_hbm, acc_sh)

    plsc.subcore_barrier()
    base = wid * EPW

    def gather(sidx, val):
        def inner(k, c2):
            iv = sidx[pl.ds(k * 16, 16)]
            val[pl.ds(k * 16, 16)] = plsc.load_gather(tab_v, [iv])
            return c2

        lax.fori_loop(0, C // 16, inner, 0, unroll=8)

    def half(jp, sidx, didx, val, sem, off):
        c0 = base + (2 * jp + off) * C

        @pl.when(jp > 0)
        def _():
            pltpu.make_async_copy(val, acc_sh.at[didx], sem).wait()

        pltpu.sync_copy(edges_hbm.at[pl.ds(c0, C)], sidx)
        pltpu.sync_copy(edges_hbm.at[pl.ds(E + c0, C)], didx)
        gather(sidx, val)
        pltpu.async_copy(val, acc_sh.at[didx], sem, add=True)

    def body(jp, carry):
        half(jp, sA, dA, vA, semA, 0)
        half(jp, sB, dB, vB, semB, 1)
        return carry

    lax.fori_loop(0, NPAIR, body, 0)
    pltpu.make_async_copy(vA, acc_sh.at[dA], semA).wait()
    pltpu.make_async_copy(vB, acc_sh.at[dB], semB).wait()
    plsc.subcore_barrier()

    @pl.when(sid == 0)
    def _():
        pltpu.sync_copy(acc_sh, out_hbm.at[cid])


@functools.partial(
    pl.kernel,
    out_type=(jax.ShapeDtypeStruct((2, NP), jnp.float32),
              jax.ShapeDtypeStruct((2, NP), jnp.float32)),
    mesh=_mesh,
    compiler_params=_cp,
    scratch_types=[
        pltpu.VMEM((NP,), jnp.float32),
        pltpu.VMEM((C2,), jnp.int32),
        pltpu.VMEM((C2,), jnp.int32),
        pltpu.VMEM((C2,), jnp.int32),
        pltpu.VMEM((C2,), jnp.int32),
        pltpu.VMEM((C2,), jnp.float32),
        pltpu.VMEM((C2,), jnp.float32),
        pltpu.VMEM((C2,), jnp.float32),
        pltpu.VMEM((C2,), jnp.float32),
        pltpu.VMEM_SHARED((NP,), jnp.float32),
        pltpu.VMEM_SHARED((NP,), jnp.float32),
        pltpu.SemaphoreType.DMA,
        pltpu.SemaphoreType.DMA,
        pltpu.SemaphoreType.DMA,
        pltpu.SemaphoreType.DMA,
    ],
)
def _sc_scatter2(edges_hbm, tab_hbm, zeros_hbm, outp_hbm, outm_hbm,
                 tab_v, sA, sB, dA, dB, vpA, vpB, vmA, vmB,
                 accp_sh, accm_sh, semPA, semMA, semPB, semMB):
    cid = lax.axis_index("c")
    sid = lax.axis_index("s")
    wid = cid * 16 + sid
    pltpu.sync_copy(tab_hbm, tab_v)

    @pl.when(sid == 0)
    def _():
        pltpu.sync_copy(zeros_hbm, accp_sh)
        pltpu.sync_copy(zeros_hbm, accm_sh)

    plsc.subcore_barrier()
    base = wid * EPW

    def gather(sidx, vp, vm):
        def inner(k, c2):
            iv = sidx[pl.ds(k * 16, 16)]
            g = plsc.load_gather(tab_v, [iv])
            vp[pl.ds(k * 16, 16)] = jnp.maximum(g, 0.0)
            vm[pl.ds(k * 16, 16)] = jnp.maximum(-g, 0.0)
            return c2

        lax.fori_loop(0, C2 // 16, inner, 0, unroll=8)

    def half_a(jp):
        c0 = base + (2 * jp) * C2
        pltpu.sync_copy(edges_hbm.at[pl.ds(c0, C2)], sA)
        pltpu.sync_copy(edges_hbm.at[pl.ds(E + c0, C2)], dA)
        gather(sA, vpA, vmA)

        @pl.when(jp > 0)
        def _():
            pltpu.make_async_copy(vpB, accp_sh.at[dB], semPB).wait()
            pltpu.make_async_copy(vmB, accm_sh.at[dB], semMB).wait()

        pltpu.async_copy(vpA, accp_sh.at[dA], semPA, add=True)
        pltpu.async_copy(vmA, accm_sh.at[dA], semMA, add=True)

    def half_b(jp):
        c1 = base + (2 * jp + 1) * C2
        pltpu.sync_copy(edges_hbm.at[pl.ds(c1, C2)], sB)
        pltpu.sync_copy(edges_hbm.at[pl.ds(E + c1, C2)], dB)
        gather(sB, vpB, vmB)
        pltpu.make_async_copy(vpA, accp_sh.at[dA], semPA).wait()
        pltpu.make_async_copy(vmA, accm_sh.at[dA], semMA).wait()
        pltpu.async_copy(vpB, accp_sh.at[dB], semPB, add=True)
        pltpu.async_copy(vmB, accm_sh.at[dB], semMB, add=True)

    def body(jp, carry):
        half_a(jp)
        half_b(jp)
        return carry

    lax.fori_loop(0, NPAIR2, body, 0)
    pltpu.make_async_copy(vpB, accp_sh.at[dB], semPB).wait()
    pltpu.make_async_copy(vmB, accm_sh.at[dB], semMB).wait()
    plsc.subcore_barrier()

    @pl.when(sid == 0)
    def _():
        pltpu.sync_copy(accp_sh, outp_hbm.at[cid])
        pltpu.sync_copy(accm_sh, outm_hbm.at[cid])


def _tc1_body(d0, d1, xr, dinv_ref, u_ref):
    deg = d0[...] + d1[...] + 1.0
    y = lax.rsqrt(deg)
    dinv = y * (1.5 - 0.5 * deg * y * y)
    dinv_ref[...] = dinv
    u_ref[...] = dinv * xr[...]


def _tc2_body(s0, s1, u, dinv, t_ref):
    t_ref[...] = dinv[...] * dinv[...] * (s0[...] + s1[...] + u[...])


def _tc3_body(p0, p1, m0, m1, t, dinv, w1, w2, b2, wfc, bfc, out_ref):
    tt = t[...]
    dv = dinv[...]
    P = dv * (p0[...] + p1[...] + jnp.maximum(tt, 0.0))
    M = dv * (m0[...] + m1[...] + jnp.maximum(-tt, 0.0))
    w1v = w1[...]
    a2 = jnp.dot(jnp.maximum(w1v, 0.0), w2[...],
                 preferred_element_type=jnp.float32)
    c2 = jnp.dot(jnp.maximum(-w1v, 0.0), w2[...],
                 preferred_element_type=jnp.float32)
    row = lax.broadcasted_iota(jnp.int32, (ROWS, 128), 0)
    col = lax.broadcasted_iota(jnp.int32, (ROWS, 128), 1)
    valid = (row * 128 + col) < N
    b2v = b2[...]
    sums = []
    for j in range(64):
        z = P * a2[0, j] + M * c2[0, j] + b2v[0, j]
        sums.append(jnp.sum(jnp.where(valid & (z > 0), z, 0.0)))
    pooled = jnp.stack(sums).reshape(1, 64) * (1.0 / N)
    out_ref[...] = jnp.dot(pooled, wfc[...],
                           preferred_element_type=jnp.float32) + bfc[...]


_tc1 = pl.pallas_call(
    _tc1_body,
    out_shape=(jax.ShapeDtypeStruct((ROWS, 128), jnp.float32),
               jax.ShapeDtypeStruct((ROWS, 128), jnp.float32)),
)

_tc2 = pl.pallas_call(
    _tc2_body,
    out_shape=jax.ShapeDtypeStruct((ROWS, 128), jnp.float32),
)

_tc3 = pl.pallas_call(
    _tc3_body,
    out_shape=jax.ShapeDtypeStruct((1, 2), jnp.float32),
)


def kernel(x, edge_index, W1, b1, W2, b2, Wfc, bfc):
    edges = edge_index.reshape(2 * E)
    zeros_np = jnp.zeros((NP,), jnp.float32)
    xp = jnp.pad(x[:, 0], (0, NP - N))

    deg_parts = _sc_degree(edges, zeros_np)
    d0 = deg_parts[0].reshape(ROWS, 128)
    d1 = deg_parts[1].reshape(ROWS, 128)
    dinv, u = _tc1(d0, d1, xp.reshape(ROWS, 128))

    s1_parts = _sc_scatter1(edges, u.reshape(NP), zeros_np)
    t = _tc2(s1_parts[0].reshape(ROWS, 128), s1_parts[1].reshape(ROWS, 128),
             u, dinv)

    p_parts, m_parts = _sc_scatter2(edges, t.reshape(NP), zeros_np)
    out = _tc3(p_parts[0].reshape(ROWS, 128), p_parts[1].reshape(ROWS, 128),
               m_parts[0].reshape(ROWS, 128), m_parts[1].reshape(ROWS, 128),
               t, dinv, W1, W2, b2.reshape(1, 64), Wfc, bfc.reshape(1, 2))
    return out
